# baseline (speedup 1.0000x reference)
import functools

import jax
import jax.numpy as jnp
from jax.experimental import pallas as pl
from jax.experimental.pallas import tpu as pltpu


def _pool_kernel(x_ref, pw_ref, o_ref, *, d):
    xv = x_ref[...]
    tr, w = xv.shape
    z = xv.reshape(tr // d, d * w)
    xs = z[:, 0:w].astype(jnp.float32)
    for j in range(1, d):
        xs = xs + z[:, j * w:(j + 1) * w]
    o_ref[...] = jnp.dot(
        xs, pw_ref[...], preferred_element_type=jnp.float32
    ).astype(o_ref.dtype)


def _avg_pool(x, d):
    B, C, H, W = x.shape
    Ho, Wo = H // d, W // d
    if H != Ho * d or W != Wo * d:
        x = x[:, :, : Ho * d, : Wo * d]
        H, W = Ho * d, Wo * d
    R = B * C * H

    a = x.reshape(R, W)

    pw = (jnp.arange(W) // d)[:, None] == jnp.arange(Wo)[None, :]
    pw = pw.astype(jnp.float32) * (1.0 / (d * d))

    tr = 8192
    while R % tr and tr > d * 8:
        tr //= 2
    grid = (R // tr,)

    itemsize = x.dtype.itemsize
    cost = pl.CostEstimate(
        flops=R * W + 2 * (R // d) * W * Wo,
        transcendentals=0,
        bytes_accessed=R * W * itemsize + W * Wo * 4 + (R // d) * Wo * itemsize,
    )

    out = pl.pallas_call(
        functools.partial(_pool_kernel, d=d),
        out_shape=jax.ShapeDtypeStruct((R // d, Wo), x.dtype),
        grid=grid,
        in_specs=[
            pl.BlockSpec((tr, W), lambda i: (i, 0)),
            pl.BlockSpec((W, Wo), lambda i: (0, 0)),
        ],
        out_specs=pl.BlockSpec((tr // d, Wo), lambda i: (i, 0)),
        compiler_params=pltpu.CompilerParams(
            dimension_semantics=("parallel",),
            vmem_limit_bytes=64 << 20,
        ),
        cost_estimate=cost,
    )(a, pw)

    return out.reshape(B, C, Ho, Wo)


def kernel(x):
    return _avg_pool(x, 2)


# speedup vs baseline: 1.0501x; 1.0501x over previous
import functools

import jax
import jax.numpy as jnp
from jax.experimental import pallas as pl
from jax.experimental.pallas import tpu as pltpu


def _pool_kernel(x_ref, pw_ref, o_ref, *, d):
    xv = x_ref[...]
    tr, w = xv.shape
    z = xv.reshape(tr // d, d * w)
    xs = z[:, 0:w].astype(jnp.float32)
    for j in range(1, d):
        xs = xs + z[:, j * w:(j + 1) * w]
    o_ref[...] = jnp.dot(
        xs, pw_ref[...], preferred_element_type=jnp.float32
    ).astype(o_ref.dtype)


def _avg_pool(x, d):
    B, C, H, W = x.shape
    Ho, Wo = H // d, W // d
    if H != Ho * d or W != Wo * d:
        x = x[:, :, : Ho * d, : Wo * d]
        H, W = Ho * d, Wo * d
    R = B * C * H

    a = x.reshape(R, W)

    pw = (jnp.arange(W) // d)[:, None] == jnp.arange(Wo)[None, :]
    pw = pw.astype(jnp.float32) * (1.0 / (d * d))

    tr = 16384
    while R % tr and tr > d * 8:
        tr //= 2
    grid = (R // tr,)

    itemsize = x.dtype.itemsize
    cost = pl.CostEstimate(
        flops=R * W + 2 * (R // d) * W * Wo,
        transcendentals=0,
        bytes_accessed=R * W * itemsize + W * Wo * 4 + (R // d) * Wo * itemsize,
    )

    out = pl.pallas_call(
        functools.partial(_pool_kernel, d=d),
        out_shape=jax.ShapeDtypeStruct((R // d, Wo), x.dtype),
        grid=grid,
        in_specs=[
            pl.BlockSpec((tr, W), lambda i: (i, 0)),
            pl.BlockSpec((W, Wo), lambda i: (0, 0)),
        ],
        out_specs=pl.BlockSpec((tr // d, Wo), lambda i: (i, 0)),
        compiler_params=pltpu.CompilerParams(
            dimension_semantics=("parallel",),
            vmem_limit_bytes=64 << 20,
        ),
        cost_estimate=cost,
    )(a, pw)

    return out.reshape(B, C, Ho, Wo)


def kernel(x):
    return _avg_pool(x, 2)


# speedup vs baseline: 1.0671x; 1.0162x over previous
import functools

import jax
import jax.numpy as jnp
import numpy as np
from jax.experimental import pallas as pl
from jax.experimental.pallas import tpu as pltpu


def _pool_kernel(x_ref, pw_ref, o_ref, *, d):
    xv = x_ref[...]
    tr, w = xv.shape
    z = xv.reshape(tr // d, d * w)
    xs = z[:, 0:w].astype(jnp.float32)
    for j in range(1, d):
        xs = xs + z[:, j * w:(j + 1) * w]
    o_ref[...] = jnp.dot(
        xs, pw_ref[...], preferred_element_type=jnp.float32
    ).astype(o_ref.dtype)


def _avg_pool(x, d):
    B, C, H, W = x.shape
    Ho, Wo = H // d, W // d
    if H != Ho * d or W != Wo * d:
        x = x[:, :, : Ho * d, : Wo * d]
        H, W = Ho * d, Wo * d
    R = B * C * H

    a = x.reshape(R, W)

    pw = (np.arange(W) // d)[:, None] == np.arange(Wo)[None, :]
    pw = jnp.asarray(pw * (1.0 / (d * d)), dtype=jnp.float32)

    tr = 16384
    while R % tr and tr > d * 8:
        tr //= 2
    grid = (R // tr,)

    itemsize = x.dtype.itemsize
    cost = pl.CostEstimate(
        flops=R * W + 2 * (R // d) * W * Wo,
        transcendentals=0,
        bytes_accessed=R * W * itemsize + W * Wo * 4 + (R // d) * Wo * itemsize,
    )

    out = pl.pallas_call(
        functools.partial(_pool_kernel, d=d),
        out_shape=jax.ShapeDtypeStruct((R // d, Wo), x.dtype),
        grid=grid,
        in_specs=[
            pl.BlockSpec((tr, W), lambda i: (i, 0)),
            pl.BlockSpec((W, Wo), lambda i: (0, 0)),
        ],
        out_specs=pl.BlockSpec((tr // d, Wo), lambda i: (i, 0)),
        compiler_params=pltpu.CompilerParams(
            dimension_semantics=("parallel",),
            vmem_limit_bytes=64 << 20,
        ),
        cost_estimate=cost,
    )(a, pw)

    return out.reshape(B, C, Ho, Wo)


def kernel(x):
    return _avg_pool(x, 2)


# speedup vs baseline: 1.0686x; 1.0014x over previous
import functools

import jax
import jax.numpy as jnp
import numpy as np
from jax.experimental import pallas as pl
from jax.experimental.pallas import tpu as pltpu


_S = 4


def _pool_kernel(*refs, d):
    pw_ref = refs[_S]
    o_ref = refs[_S + 1]
    for s in range(_S):
        xv = refs[s][...]
        tq, w = xv.shape
        z = xv.reshape(tq // d, d * w)
        xs = z[:, 0:w].astype(jnp.float32)
        for j in range(1, d):
            xs = xs + z[:, j * w:(j + 1) * w]
        o_ref[s * (tq // d):(s + 1) * (tq // d), :] = jnp.dot(
            xs, pw_ref[...], preferred_element_type=jnp.float32
        ).astype(o_ref.dtype)


def _avg_pool(x, d):
    B, C, H, W = x.shape
    Ho, Wo = H // d, W // d
    if H != Ho * d or W != Wo * d:
        x = x[:, :, : Ho * d, : Wo * d]
        H, W = Ho * d, Wo * d
    R = B * C * H

    a = x.reshape(R, W)

    pw = (np.arange(W) // d)[:, None] == np.arange(Wo)[None, :]
    pw = jnp.asarray(pw * (1.0 / (d * d)), dtype=jnp.float32)

    tr = 16384
    while R % tr and tr > d * 8:
        tr //= 2
    grid = (R // tr,)

    itemsize = x.dtype.itemsize
    cost = pl.CostEstimate(
        flops=R * W + 2 * (R // d) * W * Wo,
        transcendentals=0,
        bytes_accessed=R * W * itemsize + W * Wo * 4 + (R // d) * Wo * itemsize,
    )

    out = pl.pallas_call(
        functools.partial(_pool_kernel, d=d),
        out_shape=jax.ShapeDtypeStruct((R // d, Wo), x.dtype),
        grid=grid,
        in_specs=[
            pl.BlockSpec((tr // _S, W), (lambda s: (lambda i: (i * _S + s, 0)))(s))
            for s in range(_S)
        ]
        + [pl.BlockSpec((W, Wo), lambda i: (0, 0))],
        out_specs=pl.BlockSpec((tr // d, Wo), lambda i: (i, 0)),
        compiler_params=pltpu.CompilerParams(
            dimension_semantics=("parallel",),
            vmem_limit_bytes=64 << 20,
        ),
        cost_estimate=cost,
    )(*([a] * _S), pw)

    return out.reshape(B, C, Ho, Wo)


def kernel(x):
    return _avg_pool(x, 2)


# speedup vs baseline: 1.0720x; 1.0032x over previous
import functools

import jax
import jax.numpy as jnp
import numpy as np
from jax.experimental import pallas as pl
from jax.experimental.pallas import tpu as pltpu


def _pool_kernel(x_ref, pw_ref, o_ref, *, d):
    xv = x_ref[...]
    tr, w = xv.shape
    z = xv.reshape(tr // d, d * w)
    xs = z[:, 0:w].astype(jnp.float32)
    for j in range(1, d):
        xs = xs + z[:, j * w:(j + 1) * w]
    o_ref[...] = jnp.dot(
        xs, pw_ref[...], preferred_element_type=jnp.float32
    ).astype(o_ref.dtype)


def _avg_pool(x, d):
    B, C, H, W = x.shape
    Ho, Wo = H // d, W // d
    if H != Ho * d or W != Wo * d:
        x = x[:, :, : Ho * d, : Wo * d]
        H, W = Ho * d, Wo * d
    R = B * C * H

    a = x.reshape(R, W)

    pw = (np.arange(W) // d)[:, None] == np.arange(Wo)[None, :]
    pw = jnp.asarray(pw * (1.0 / (d * d)), dtype=jnp.float32)

    tr = 16384
    while R % tr and tr > d * 8:
        tr //= 2
    grid = (R // tr,)

    itemsize = x.dtype.itemsize
    cost = pl.CostEstimate(
        flops=R * W + 2 * (R // d) * W * Wo,
        transcendentals=0,
        bytes_accessed=R * W * itemsize + W * Wo * 4 + (R // d) * Wo * itemsize,
    )

    out = pl.pallas_call(
        functools.partial(_pool_kernel, d=d),
        out_shape=jax.ShapeDtypeStruct((R // d, Wo), x.dtype),
        grid=grid,
        in_specs=[
            pl.BlockSpec((tr, W), lambda i: (i, 0)),
            pl.BlockSpec((W, Wo), lambda i: (0, 0)),
        ],
        out_specs=pl.BlockSpec((tr // d, Wo), lambda i: (i, 0)),
        compiler_params=pltpu.CompilerParams(
            dimension_semantics=("parallel",),
            vmem_limit_bytes=64 << 20,
        ),
        cost_estimate=cost,
    )(a, pw)

    return out.reshape(B, C, Ho, Wo)


def kernel(x):
    return _avg_pool(x, 2)
